# Initial kernel scaffold; baseline (speedup 1.0000x reference)
import jax
import jax.numpy as jnp
from jax import lax
from jax.experimental import pallas as pl
from jax.experimental.pallas import tpu as pltpu
from jax.experimental.pallas import tpu_sc as plsc

N_NODES = 100000
N_EDGES = 6400000
N_STEPS = 5
R = 0.025
THRESHOLD = 0.001378
B = 0.001
NOISE_SPARSITY = 1.5
NOISE_STD = 0.3
TAU = 0.01
DT = 0.0001

NT = 16
SL = 6400
N_PAD = NT * SL
CHUNK = 2048
K = CHUNK // 128
CPT = 196
E_PAD = NT * CPT * CHUNK
ROWS_PER_TILE = CPT * K


def _body(act_hbm, src_hbm, dst_hbm, w_hbm, bt_hbm, out_hbm,
          act_sp, acc_sp, src_v, dst_v, w_v, val_v,
          ybuf, bbuf, sbuf, abuf, zbuf, gsem, ssem):
    core = lax.axis_index("c")
    sid = lax.axis_index("s")

    @pl.when(core == 0)
    def _():
        nslice = pl.ds(sid * SL, SL)

        def zinit(i, carry):
            zbuf[pl.ds(i * 16, 16)] = jnp.zeros((16,), jnp.float32)
            return carry
        lax.fori_loop(0, SL // 16, zinit, 0)

        pltpu.sync_copy(act_hbm.at[nslice], act_sp.at[nslice])
        pltpu.sync_copy(zbuf, acc_sp.at[nslice])
        plsc.subcore_barrier()

        row_base = sid * ROWS_PER_TILE
        for t in range(N_STEPS):
            def chunk_body(c, carry):
                r0 = row_base + c * K
                pltpu.sync_copy(src_hbm.at[pl.ds(r0, K)], src_v)
                pltpu.sync_copy(dst_hbm.at[pl.ds(r0, K)], dst_v)
                pltpu.sync_copy(w_hbm.at[pl.ds(r0, K)], w_v)
                gds = [pltpu.async_copy(act_sp.at[src_v.at[j]],
                                        val_v.at[j], gsem)
                       for j in range(K)]
                for d in gds:
                    d.wait()
                for j in range(K):
                    for m in range(8):
                        s = pl.ds(m * 16, 16)
                        val_v[j, s] = val_v[j, s] * w_v[j, s]
                sds = [pltpu.async_copy(val_v.at[j],
                                        acc_sp.at[dst_v.at[j]],
                                        ssem, add=True)
                       for j in range(K)]
                for d in sds:
                    d.wait()
                return carry
            lax.fori_loop(0, CPT, chunk_body, 0)
            plsc.subcore_barrier()

            pltpu.sync_copy(acc_sp.at[nslice], ybuf)
            pltpu.sync_copy(bt_hbm.at[t, nslice], bbuf)

            def ew(i, carry):
                s = pl.ds(i * 16, 16)
                y = ybuf[s]
                l = R * y + bbuf[s]
                spk = jnp.where(l > THRESHOLD,
                                jnp.float32(1.0), jnp.float32(0.0))
                sbuf[s] = spk
                abuf[s] = y + spk - y / TAU * DT
                return carry
            lax.fori_loop(0, SL // 16, ew, 0)

            pltpu.sync_copy(sbuf, out_hbm.at[t, nslice])
            pltpu.sync_copy(abuf, act_sp.at[nslice])
            pltpu.sync_copy(zbuf, acc_sp.at[nslice])
            plsc.subcore_barrier()


_sc_call = pl.kernel(
    _body,
    out_type=jax.ShapeDtypeStruct((N_STEPS, N_PAD), jnp.float32),
    mesh=plsc.VectorSubcoreMesh(core_axis_name="c", subcore_axis_name="s"),
    scratch_types=[
        pltpu.VMEM_SHARED((N_PAD,), jnp.float32),
        pltpu.VMEM_SHARED((N_PAD,), jnp.float32),
        pltpu.VMEM((K, 128), jnp.int32),
        pltpu.VMEM((K, 128), jnp.int32),
        pltpu.VMEM((K, 128), jnp.float32),
        pltpu.VMEM((K, 128), jnp.float32),
        pltpu.VMEM((SL,), jnp.float32),
        pltpu.VMEM((SL,), jnp.float32),
        pltpu.VMEM((SL,), jnp.float32),
        pltpu.VMEM((SL,), jnp.float32),
        pltpu.VMEM((SL,), jnp.float32),
        pltpu.SemaphoreType.DMA,
        pltpu.SemaphoreType.DMA,
    ],
)


def kernel(activation, weights, edge_index):
    act = jnp.pad(activation, (0, N_PAD - N_NODES))
    src = edge_index[0]
    dst = edge_index[1]
    pad = E_PAD - N_EDGES
    fill = (jnp.arange(pad, dtype=jnp.int32) * 997) % N_NODES
    src2 = jnp.concatenate([src, fill]).reshape(E_PAD // 128, 128)
    dst2 = jnp.concatenate([dst, fill]).reshape(E_PAD // 128, 128)
    w2 = jnp.concatenate(
        [weights, jnp.zeros((pad,), jnp.float32)]).reshape(E_PAD // 128, 128)

    noise_key = jax.random.key(42)
    bts = []
    for t in range(N_STEPS):
        kt = jax.random.fold_in(noise_key, t)
        ka, kb = jax.random.split(kt)
        noise = NOISE_STD * jax.random.normal(ka, (N_NODES,), jnp.float32)
        filt = (jax.random.normal(kb, (N_NODES,), jnp.float32)
                > NOISE_SPARSITY).astype(jnp.float32)
        bts.append(B * (1.0 + noise * filt))
    bt = jnp.pad(jnp.stack(bts), ((0, 0), (0, N_PAD - N_NODES)))

    out = _sc_call(act, src2, dst2, w2, bt)
    return out.T[:N_NODES]


# speedup vs baseline: 87.2536x; 87.2536x over previous
import jax
import jax.numpy as jnp
from jax import lax
from jax.experimental import pallas as pl
from jax.experimental.pallas import tpu as pltpu
from jax.experimental.pallas import tpu_sc as plsc

N_NODES = 100000
N_EDGES = 6400000
N_STEPS = 5
R = 0.025
THRESHOLD = 0.001378
B = 0.001
NOISE_SPARSITY = 1.5
NOISE_STD = 0.3
TAU = 0.01
DT = 0.0001

NT = 16
SL = 6400
N_PAD = NT * SL
CHUNK = 2048
K = CHUNK // 128
CPT = 196
E_PAD = NT * CPT * CHUNK
ROWS_PER_TILE = CPT * K


def _body(act_hbm, src_hbm, dst_hbm, w_hbm, bt_hbm, out_hbm,
          act_sp, acc_sp, src_v, dst_v, w_v, val_v,
          ybuf, bbuf, sbuf, abuf, zbuf, gsem, ssem):
    core = lax.axis_index("c")
    sid = lax.axis_index("s")

    @pl.when(core == 0)
    def _():
        nslice = pl.ds(sid * SL, SL)

        def zinit(i, carry):
            zbuf[pl.ds(i * 16, 16)] = jnp.zeros((16,), jnp.float32)
            return carry
        lax.fori_loop(0, SL // 16, zinit, 0)

        pltpu.sync_copy(act_hbm.at[nslice], act_sp.at[nslice])
        pltpu.sync_copy(zbuf, acc_sp.at[nslice])
        plsc.subcore_barrier()

        row_base = sid * ROWS_PER_TILE
        for t in range(N_STEPS):
            def chunk_body(c, carry):
                r0 = row_base + c * K
                pltpu.sync_copy(src_hbm.at[pl.ds(r0, K)], src_v)
                pltpu.sync_copy(dst_hbm.at[pl.ds(r0, K)], dst_v)
                pltpu.sync_copy(w_hbm.at[pl.ds(r0, K)], w_v)
                gds = [pltpu.async_copy(act_sp.at[src_v.at[j]],
                                        val_v.at[j], gsem)
                       for j in range(K)]
                for d in gds:
                    d.wait()
                for j in range(K):
                    for m in range(8):
                        s = pl.ds(m * 16, 16)
                        val_v[j, s] = val_v[j, s] * w_v[j, s]
                sds = [pltpu.async_copy(val_v.at[j],
                                        acc_sp.at[dst_v.at[j]],
                                        ssem, add=True)
                       for j in range(K)]
                for d in sds:
                    d.wait()
                return carry
            lax.fori_loop(0, CPT, chunk_body, 0)
            plsc.subcore_barrier()

            pltpu.sync_copy(acc_sp.at[nslice], ybuf)
            pltpu.sync_copy(bt_hbm.at[pl.ds(t * N_PAD + sid * SL, SL)], bbuf)

            def ew(i, carry):
                s = pl.ds(i * 16, 16)
                y = ybuf[s]
                l = R * y + bbuf[s]
                spk = jnp.where(l > THRESHOLD,
                                jnp.float32(1.0), jnp.float32(0.0))
                sbuf[s] = spk
                abuf[s] = y + spk - y / TAU * DT
                return carry
            lax.fori_loop(0, SL // 16, ew, 0)

            pltpu.sync_copy(sbuf, out_hbm.at[pl.ds(t * N_PAD + sid * SL, SL)])
            pltpu.sync_copy(abuf, act_sp.at[nslice])
            pltpu.sync_copy(zbuf, acc_sp.at[nslice])
            plsc.subcore_barrier()


_sc_call = pl.kernel(
    _body,
    out_type=jax.ShapeDtypeStruct((N_STEPS * N_PAD,), jnp.float32),
    mesh=plsc.VectorSubcoreMesh(core_axis_name="c", subcore_axis_name="s"),
    scratch_types=[
        pltpu.VMEM_SHARED((N_PAD,), jnp.float32),
        pltpu.VMEM_SHARED((N_PAD,), jnp.float32),
        pltpu.VMEM((K, 128), jnp.int32),
        pltpu.VMEM((K, 128), jnp.int32),
        pltpu.VMEM((K, 128), jnp.float32),
        pltpu.VMEM((K, 128), jnp.float32),
        pltpu.VMEM((SL,), jnp.float32),
        pltpu.VMEM((SL,), jnp.float32),
        pltpu.VMEM((SL,), jnp.float32),
        pltpu.VMEM((SL,), jnp.float32),
        pltpu.VMEM((SL,), jnp.float32),
        pltpu.SemaphoreType.DMA,
        pltpu.SemaphoreType.DMA,
    ],
)


def kernel(activation, weights, edge_index):
    act = jnp.pad(activation, (0, N_PAD - N_NODES))
    src = edge_index[0]
    dst = edge_index[1]
    pad = E_PAD - N_EDGES
    fill = (jnp.arange(pad, dtype=jnp.int32) * 997) % N_NODES
    src2 = jnp.concatenate([src, fill]).reshape(E_PAD // 128, 128)
    dst2 = jnp.concatenate([dst, fill]).reshape(E_PAD // 128, 128)
    w2 = jnp.concatenate(
        [weights, jnp.zeros((pad,), jnp.float32)]).reshape(E_PAD // 128, 128)

    noise_key = jax.random.key(42)
    bts = []
    for t in range(N_STEPS):
        kt = jax.random.fold_in(noise_key, t)
        ka, kb = jax.random.split(kt)
        noise = NOISE_STD * jax.random.normal(ka, (N_NODES,), jnp.float32)
        filt = (jax.random.normal(kb, (N_NODES,), jnp.float32)
                > NOISE_SPARSITY).astype(jnp.float32)
        bts.append(B * (1.0 + noise * filt))
    bt = jnp.pad(jnp.stack(bts), ((0, 0), (0, N_PAD - N_NODES))).reshape(-1)

    out = _sc_call(act, src2, dst2, w2, bt)
    return out.reshape(N_STEPS, N_PAD).T[:N_NODES]


# speedup vs baseline: 149.1198x; 1.7090x over previous
import jax
import jax.numpy as jnp
from jax import lax
from jax.experimental import pallas as pl
from jax.experimental.pallas import tpu as pltpu
from jax.experimental.pallas import tpu_sc as plsc

N_NODES = 100000
N_EDGES = 6400000
N_STEPS = 5
R = 0.025
THRESHOLD = 0.001378
B = 0.001
NOISE_SPARSITY = 1.5
NOISE_STD = 0.3
TAU = 0.01
DT = 0.0001

NT = 16
SL = 6400
N_PAD = NT * SL
CHUNK = 4096
K = CHUNK // 128
CPT = 98
E_PAD = NT * CPT * CHUNK
ROWS_PER_TILE = CPT * K


def _body(act_hbm, src_hbm, dst_hbm, w_hbm, bt_hbm, out_hbm,
          act_sp, acc_sp, src_v, dst_v, w_v, val_v,
          ybuf, bbuf, sbuf, abuf, zbuf, lsem, gsem, ssem):
    core = lax.axis_index("c")
    sid = lax.axis_index("s")

    @pl.when(core == 0)
    def _():
        nslice = pl.ds(sid * SL, SL)

        def zinit(i, carry):
            zbuf[pl.ds(i * 16, 16)] = jnp.zeros((16,), jnp.float32)
            return carry
        lax.fori_loop(0, SL // 16, zinit, 0)

        pltpu.sync_copy(act_hbm.at[nslice], act_sp.at[nslice])
        pltpu.sync_copy(zbuf, acc_sp.at[nslice])
        plsc.subcore_barrier()

        row_base = sid * ROWS_PER_TILE

        def fire_lin(c, p):
            r0 = row_base + c * K
            pltpu.async_copy(src_hbm.at[pl.ds(r0, K)], src_v.at[p], lsem)
            pltpu.async_copy(dst_hbm.at[pl.ds(r0, K)], dst_v.at[p], lsem)
            pltpu.async_copy(w_hbm.at[pl.ds(r0, K)], w_v.at[p], lsem)

        def wait_lin(c, p):
            r0 = row_base + c * K
            pltpu.make_async_copy(src_hbm.at[pl.ds(r0, K)],
                                  src_v.at[p], lsem).wait()
            pltpu.make_async_copy(dst_hbm.at[pl.ds(r0, K)],
                                  dst_v.at[p], lsem).wait()
            pltpu.make_async_copy(w_hbm.at[pl.ds(r0, K)],
                                  w_v.at[p], lsem).wait()

        def wait_scat(p):
            for j in range(K):
                pltpu.make_async_copy(val_v.at[p, j],
                                      acc_sp.at[dst_v.at[p, j]],
                                      ssem).wait()

        def step_body(t, carry0):
            fire_lin(0, 0)

            def group_body(g, carry):
                for b in range(2):
                    p, q = b, 1 - b
                    c = 2 * g + b
                    wait_lin(c, p)
                    gds = [pltpu.async_copy(act_sp.at[src_v.at[p, j]],
                                            val_v.at[p, j], gsem)
                           for j in range(K)]
                    if b == 0:
                        @pl.when(g > 0)
                        def _():
                            wait_scat(q)
                        fire_lin(c + 1, q)
                    else:
                        wait_scat(q)

                        @pl.when(g < CPT // 2 - 1)
                        def _():
                            fire_lin(c + 1, q)
                    for d in gds:
                        d.wait()

                    def mul_body(i, mc):
                        j = i // 8
                        s = pl.ds((i % 8) * 16, 16)
                        val_v[p, j, s] = val_v[p, j, s] * w_v[p, j, s]
                        return mc
                    lax.fori_loop(0, K * 8, mul_body, 0)
                    for j in range(K):
                        pltpu.async_copy(val_v.at[p, j],
                                         acc_sp.at[dst_v.at[p, j]],
                                         ssem, add=True)
                return carry
            lax.fori_loop(0, CPT // 2, group_body, 0)
            wait_scat(1)
            plsc.subcore_barrier()

            pltpu.sync_copy(acc_sp.at[nslice], ybuf)
            pltpu.sync_copy(bt_hbm.at[pl.ds(t * N_PAD + sid * SL, SL)], bbuf)

            def ew(i, carry):
                s = pl.ds(i * 16, 16)
                y = ybuf[s]
                l = R * y + bbuf[s]
                spk = jnp.where(l > THRESHOLD,
                                jnp.float32(1.0), jnp.float32(0.0))
                sbuf[s] = spk
                abuf[s] = y + spk - y / TAU * DT
                return carry
            lax.fori_loop(0, SL // 16, ew, 0)

            pltpu.sync_copy(sbuf, out_hbm.at[pl.ds(t * N_PAD + sid * SL, SL)])
            pltpu.sync_copy(abuf, act_sp.at[nslice])
            pltpu.sync_copy(zbuf, acc_sp.at[nslice])
            plsc.subcore_barrier()
            return carry0
        lax.fori_loop(0, N_STEPS, step_body, 0)


_sc_call = pl.kernel(
    _body,
    out_type=jax.ShapeDtypeStruct((N_STEPS * N_PAD,), jnp.float32),
    mesh=plsc.VectorSubcoreMesh(core_axis_name="c", subcore_axis_name="s"),
    scratch_types=[
        pltpu.VMEM_SHARED((N_PAD,), jnp.float32),
        pltpu.VMEM_SHARED((N_PAD,), jnp.float32),
        pltpu.VMEM((2, K, 128), jnp.int32),
        pltpu.VMEM((2, K, 128), jnp.int32),
        pltpu.VMEM((2, K, 128), jnp.float32),
        pltpu.VMEM((2, K, 128), jnp.float32),
        pltpu.VMEM((SL,), jnp.float32),
        pltpu.VMEM((SL,), jnp.float32),
        pltpu.VMEM((SL,), jnp.float32),
        pltpu.VMEM((SL,), jnp.float32),
        pltpu.VMEM((SL,), jnp.float32),
        pltpu.SemaphoreType.DMA,
        pltpu.SemaphoreType.DMA,
        pltpu.SemaphoreType.DMA,
    ],
)


def kernel(activation, weights, edge_index):
    act = jnp.pad(activation, (0, N_PAD - N_NODES))
    src = edge_index[0]
    dst = edge_index[1]
    pad = E_PAD - N_EDGES
    fill = (jnp.arange(pad, dtype=jnp.int32) * 997) % N_NODES
    src2 = jnp.concatenate([src, fill]).reshape(E_PAD // 128, 128)
    dst2 = jnp.concatenate([dst, fill]).reshape(E_PAD // 128, 128)
    w2 = jnp.concatenate(
        [weights, jnp.zeros((pad,), jnp.float32)]).reshape(E_PAD // 128, 128)

    noise_key = jax.random.key(42)
    bts = []
    for t in range(N_STEPS):
        kt = jax.random.fold_in(noise_key, t)
        ka, kb = jax.random.split(kt)
        noise = NOISE_STD * jax.random.normal(ka, (N_NODES,), jnp.float32)
        filt = (jax.random.normal(kb, (N_NODES,), jnp.float32)
                > NOISE_SPARSITY).astype(jnp.float32)
        bts.append(B * (1.0 + noise * filt))
    bt = jnp.pad(jnp.stack(bts), ((0, 0), (0, N_PAD - N_NODES))).reshape(-1)

    out = _sc_call(act, src2, dst2, w2, bt)
    return out.reshape(N_STEPS, N_PAD).T[:N_NODES]


# speedup vs baseline: 277.7129x; 1.8623x over previous
import jax
import jax.numpy as jnp
from jax import lax
from jax.experimental import pallas as pl
from jax.experimental.pallas import tpu as pltpu
from jax.experimental.pallas import tpu_sc as plsc

N_NODES = 100000
N_EDGES = 6400000
N_STEPS = 5
R = 0.025
THRESHOLD = 0.001378
B = 0.001
NOISE_SPARSITY = 1.5
NOISE_STD = 0.3
TAU = 0.01
DT = 0.0001

NC = 2
NT = 16
SL = 6400
N_PAD = NT * SL
SL2 = N_PAD // (NC * NT)
CHUNK = 2048
K = CHUNK // 128
CPT = 98
E_PAD = NC * NT * CPT * CHUNK
ROWS_PER_TILE = CPT * K


def _edge_body(act_hbm, src_hbm, dst_hbm, w_hbm, part_hbm,
               act_sp, acc_sp, src_v, dst_v, w_v, val_v, zbuf,
               lsem, gsem, ssem):
    core = lax.axis_index("c")
    sid = lax.axis_index("s")
    nslice = pl.ds(sid * SL, SL)

    def zinit(i, carry):
        zbuf[pl.ds(i * 16, 16)] = jnp.zeros((16,), jnp.float32)
        return carry
    lax.fori_loop(0, SL // 16, zinit, 0)

    pltpu.sync_copy(act_hbm.at[nslice], act_sp.at[nslice])
    pltpu.sync_copy(zbuf, acc_sp.at[nslice])
    plsc.subcore_barrier()

    row_base = (core * NT + sid) * ROWS_PER_TILE

    def fire_lin(c, p):
        r0 = row_base + c * K
        pltpu.async_copy(src_hbm.at[pl.ds(r0, K)], src_v.at[p], lsem)
        pltpu.async_copy(dst_hbm.at[pl.ds(r0, K)], dst_v.at[p], lsem)
        pltpu.async_copy(w_hbm.at[pl.ds(r0, K)], w_v.at[p], lsem)

    def wait_lin(c, p):
        r0 = row_base + c * K
        pltpu.make_async_copy(src_hbm.at[pl.ds(r0, K)],
                              src_v.at[p], lsem).wait()
        pltpu.make_async_copy(dst_hbm.at[pl.ds(r0, K)],
                              dst_v.at[p], lsem).wait()
        pltpu.make_async_copy(w_hbm.at[pl.ds(r0, K)],
                              w_v.at[p], lsem).wait()

    def wait_scat(p):
        for j in range(K):
            pltpu.make_async_copy(val_v.at[p, j],
                                  acc_sp.at[dst_v.at[p, j]],
                                  ssem).wait()

    fire_lin(0, 0)

    def group_body(g, carry):
        for b in range(2):
            p, q = b, 1 - b
            c = 2 * g + b
            wait_lin(c, p)
            gds = [pltpu.async_copy(act_sp.at[src_v.at[p, j]],
                                    val_v.at[p, j], gsem)
                   for j in range(K)]
            if b == 0:
                @pl.when(g > 0)
                def _():
                    wait_scat(q)
                fire_lin(c + 1, q)
            else:
                wait_scat(q)

                @pl.when(g < CPT // 2 - 1)
                def _():
                    fire_lin(c + 1, q)
            for d in gds:
                d.wait()

            def mul_body(i, mc):
                j = i // 8
                s = pl.ds((i % 8) * 16, 16)
                val_v[p, j, s] = val_v[p, j, s] * w_v[p, j, s]
                return mc
            lax.fori_loop(0, K * 8, mul_body, 0)
            for j in range(K):
                pltpu.async_copy(val_v.at[p, j],
                                 acc_sp.at[dst_v.at[p, j]],
                                 ssem, add=True)
        return carry
    lax.fori_loop(0, CPT // 2, group_body, 0)
    wait_scat(1)
    plsc.subcore_barrier()
    pltpu.sync_copy(acc_sp.at[nslice],
                    part_hbm.at[pl.ds(core * N_PAD + sid * SL, SL)])


def _elem_body(part_hbm, bt_hbm, out_hbm, act_hbm,
               y0buf, y1buf, bbuf, sbuf, abuf):
    core = lax.axis_index("c")
    sid = lax.axis_index("s")
    wid = sid * NC + core
    base = wid * SL2

    pltpu.sync_copy(part_hbm.at[pl.ds(base, SL2)], y0buf)
    pltpu.sync_copy(part_hbm.at[pl.ds(N_PAD + base, SL2)], y1buf)
    pltpu.sync_copy(bt_hbm.at[pl.ds(base, SL2)], bbuf)

    def ew(i, carry):
        s = pl.ds(i * 16, 16)
        y = y0buf[s] + y1buf[s]
        l = R * y + bbuf[s]
        spk = jnp.where(l > THRESHOLD, jnp.float32(1.0), jnp.float32(0.0))
        sbuf[s] = spk
        abuf[s] = y + spk - y / TAU * DT
        return carry
    lax.fori_loop(0, SL2 // 16, ew, 0)

    pltpu.sync_copy(sbuf, out_hbm.at[pl.ds(base, SL2)])
    pltpu.sync_copy(abuf, act_hbm.at[pl.ds(base, SL2)])


_mesh = plsc.VectorSubcoreMesh(core_axis_name="c", subcore_axis_name="s")

_edge_call = pl.kernel(
    _edge_body,
    out_type=jax.ShapeDtypeStruct((NC * N_PAD,), jnp.float32),
    mesh=_mesh,
    scratch_types=[
        pltpu.VMEM_SHARED((N_PAD,), jnp.float32),
        pltpu.VMEM_SHARED((N_PAD,), jnp.float32),
        pltpu.VMEM((2, K, 128), jnp.int32),
        pltpu.VMEM((2, K, 128), jnp.int32),
        pltpu.VMEM((2, K, 128), jnp.float32),
        pltpu.VMEM((2, K, 128), jnp.float32),
        pltpu.VMEM((SL,), jnp.float32),
        pltpu.SemaphoreType.DMA,
        pltpu.SemaphoreType.DMA,
        pltpu.SemaphoreType.DMA,
    ],
)

_elem_call = pl.kernel(
    _elem_body,
    out_type=(jax.ShapeDtypeStruct((N_PAD,), jnp.float32),
              jax.ShapeDtypeStruct((N_PAD,), jnp.float32)),
    mesh=_mesh,
    scratch_types=[
        pltpu.VMEM((SL2,), jnp.float32),
        pltpu.VMEM((SL2,), jnp.float32),
        pltpu.VMEM((SL2,), jnp.float32),
        pltpu.VMEM((SL2,), jnp.float32),
        pltpu.VMEM((SL2,), jnp.float32),
    ],
)


def kernel(activation, weights, edge_index):
    act = jnp.pad(activation, (0, N_PAD - N_NODES))
    src = edge_index[0]
    dst = edge_index[1]
    pad = E_PAD - N_EDGES
    fill = (jnp.arange(pad, dtype=jnp.int32) * 997) % N_NODES
    src2 = jnp.concatenate([src, fill]).reshape(E_PAD // 128, 128)
    dst2 = jnp.concatenate([dst, fill]).reshape(E_PAD // 128, 128)
    w2 = jnp.concatenate(
        [weights, jnp.zeros((pad,), jnp.float32)]).reshape(E_PAD // 128, 128)

    noise_key = jax.random.key(42)
    spikes = []
    for t in range(N_STEPS):
        kt = jax.random.fold_in(noise_key, t)
        ka, kb = jax.random.split(kt)
        noise = NOISE_STD * jax.random.normal(ka, (N_NODES,), jnp.float32)
        filt = (jax.random.normal(kb, (N_NODES,), jnp.float32)
                > NOISE_SPARSITY).astype(jnp.float32)
        bt = jnp.pad(B * (1.0 + noise * filt), (0, N_PAD - N_NODES))
        part = _edge_call(act, src2, dst2, w2)
        spk, act = _elem_call(part, bt)
        spikes.append(spk[:N_NODES])
    return jnp.stack(spikes, axis=1)


# speedup vs baseline: 309.6493x; 1.1150x over previous
import jax
import jax.numpy as jnp
from jax import lax
from jax.experimental import pallas as pl
from jax.experimental.pallas import tpu as pltpu
from jax.experimental.pallas import tpu_sc as plsc

N_NODES = 100000
N_EDGES = 6400000
N_STEPS = 5
R = 0.025
THRESHOLD = 0.001378
B = 0.001
NOISE_SPARSITY = 1.5
NOISE_STD = 0.3
TAU = 0.01
DT = 0.0001

NC = 2
NT = 16
SL = 6400
N_PAD = NT * SL
SL2 = N_PAD // (NC * NT)
CHUNK = 2048
K = CHUNK // 128
CPT = 98
E_PAD = NC * NT * CPT * CHUNK
ROWS_PER_TILE = CPT * K


def _edge_body(act_hbm, src_hbm, dst_hbm, w_hbm, part_hbm,
               act_sp, acc_sp, src_v, dst_v, w_v, val_v, zbuf,
               lsem, gsem, ssem):
    core = lax.axis_index("c")
    sid = lax.axis_index("s")
    nslice = pl.ds(sid * SL, SL)

    def zinit(i, carry):
        zbuf[pl.ds(i * 16, 16)] = jnp.zeros((16,), jnp.float32)
        return carry
    lax.fori_loop(0, SL // 16, zinit, 0)

    pltpu.sync_copy(act_hbm.at[nslice], act_sp.at[nslice])
    pltpu.sync_copy(zbuf, acc_sp.at[nslice])
    plsc.subcore_barrier()

    row_base = (core * NT + sid) * ROWS_PER_TILE

    def fire_lin(c, p):
        r0 = row_base + c * K
        pltpu.async_copy(src_hbm.at[pl.ds(r0, K)], src_v.at[p], lsem)
        pltpu.async_copy(dst_hbm.at[pl.ds(r0, K)], dst_v.at[p], lsem)
        pltpu.async_copy(w_hbm.at[pl.ds(r0, K)], w_v.at[p], lsem)

    def wait_lin(c, p):
        r0 = row_base + c * K
        pltpu.make_async_copy(src_hbm.at[pl.ds(r0, K)],
                              src_v.at[p], lsem).wait()
        pltpu.make_async_copy(dst_hbm.at[pl.ds(r0, K)],
                              dst_v.at[p], lsem).wait()
        pltpu.make_async_copy(w_hbm.at[pl.ds(r0, K)],
                              w_v.at[p], lsem).wait()

    def wait_scat(p):
        for j in range(K):
            pltpu.make_async_copy(val_v.at[p, j],
                                  acc_sp.at[dst_v.at[p, j]],
                                  ssem).wait()

    fire_lin(0, 0)

    def group_body(g, carry):
        for b in range(2):
            p, q = b, 1 - b
            c = 2 * g + b
            wait_lin(c, p)
            gds = [pltpu.async_copy(act_sp.at[src_v.at[p, j]],
                                    val_v.at[p, j], gsem)
                   for j in range(K)]
            if b == 0:
                @pl.when(g > 0)
                def _():
                    wait_scat(q)
                fire_lin(c + 1, q)
            else:
                wait_scat(q)

                @pl.when(g < CPT // 2 - 1)
                def _():
                    fire_lin(c + 1, q)
            for d in gds:
                d.wait()

            for j in range(K):
                for m in range(8):
                    s = pl.ds(m * 16, 16)
                    val_v[p, j, s] = val_v[p, j, s] * w_v[p, j, s]
            for j in range(K):
                pltpu.async_copy(val_v.at[p, j],
                                 acc_sp.at[dst_v.at[p, j]],
                                 ssem, add=True)
        return carry
    lax.fori_loop(0, CPT // 2, group_body, 0)
    wait_scat(1)
    plsc.subcore_barrier()
    pltpu.sync_copy(acc_sp.at[nslice],
                    part_hbm.at[pl.ds(core * N_PAD + sid * SL, SL)])


def _elem_body(part_hbm, bt_hbm, out_hbm, act_hbm,
               y0buf, y1buf, bbuf, sbuf, abuf):
    core = lax.axis_index("c")
    sid = lax.axis_index("s")
    wid = sid * NC + core
    base = wid * SL2

    pltpu.sync_copy(part_hbm.at[pl.ds(base, SL2)], y0buf)
    pltpu.sync_copy(part_hbm.at[pl.ds(N_PAD + base, SL2)], y1buf)
    pltpu.sync_copy(bt_hbm.at[pl.ds(base, SL2)], bbuf)

    def ew(i, carry):
        s = pl.ds(i * 16, 16)
        y = y0buf[s] + y1buf[s]
        l = R * y + bbuf[s]
        spk = jnp.where(l > THRESHOLD, jnp.float32(1.0), jnp.float32(0.0))
        sbuf[s] = spk
        abuf[s] = y + spk - y / TAU * DT
        return carry
    lax.fori_loop(0, SL2 // 16, ew, 0)

    pltpu.sync_copy(sbuf, out_hbm.at[pl.ds(base, SL2)])
    pltpu.sync_copy(abuf, act_hbm.at[pl.ds(base, SL2)])


_mesh = plsc.VectorSubcoreMesh(core_axis_name="c", subcore_axis_name="s")

_edge_call = pl.kernel(
    _edge_body,
    out_type=jax.ShapeDtypeStruct((NC * N_PAD,), jnp.float32),
    mesh=_mesh,
    scratch_types=[
        pltpu.VMEM_SHARED((N_PAD,), jnp.float32),
        pltpu.VMEM_SHARED((N_PAD,), jnp.float32),
        pltpu.VMEM((2, K, 128), jnp.int32),
        pltpu.VMEM((2, K, 128), jnp.int32),
        pltpu.VMEM((2, K, 128), jnp.float32),
        pltpu.VMEM((2, K, 128), jnp.float32),
        pltpu.VMEM((SL,), jnp.float32),
        pltpu.SemaphoreType.DMA,
        pltpu.SemaphoreType.DMA,
        pltpu.SemaphoreType.DMA,
    ],
)

_elem_call = pl.kernel(
    _elem_body,
    out_type=(jax.ShapeDtypeStruct((N_PAD,), jnp.float32),
              jax.ShapeDtypeStruct((N_PAD,), jnp.float32)),
    mesh=_mesh,
    scratch_types=[
        pltpu.VMEM((SL2,), jnp.float32),
        pltpu.VMEM((SL2,), jnp.float32),
        pltpu.VMEM((SL2,), jnp.float32),
        pltpu.VMEM((SL2,), jnp.float32),
        pltpu.VMEM((SL2,), jnp.float32),
    ],
)


def kernel(activation, weights, edge_index):
    act = jnp.pad(activation, (0, N_PAD - N_NODES))
    src = edge_index[0]
    dst = edge_index[1]
    pad = E_PAD - N_EDGES
    fill = (jnp.arange(pad, dtype=jnp.int32) * 997) % N_NODES
    src2 = jnp.concatenate([src, fill]).reshape(E_PAD // 128, 128)
    dst2 = jnp.concatenate([dst, fill]).reshape(E_PAD // 128, 128)
    w2 = jnp.concatenate(
        [weights, jnp.zeros((pad,), jnp.float32)]).reshape(E_PAD // 128, 128)

    noise_key = jax.random.key(42)
    spikes = []
    for t in range(N_STEPS):
        kt = jax.random.fold_in(noise_key, t)
        ka, kb = jax.random.split(kt)
        noise = NOISE_STD * jax.random.normal(ka, (N_NODES,), jnp.float32)
        filt = (jax.random.normal(kb, (N_NODES,), jnp.float32)
                > NOISE_SPARSITY).astype(jnp.float32)
        bt = jnp.pad(B * (1.0 + noise * filt), (0, N_PAD - N_NODES))
        part = _edge_call(act, src2, dst2, w2)
        spk, act = _elem_call(part, bt)
        spikes.append(spk[:N_NODES])
    return jnp.stack(spikes, axis=1)


# speedup vs baseline: 310.5173x; 1.0028x over previous
import jax
import jax.numpy as jnp
from jax import lax
from jax.experimental import pallas as pl
from jax.experimental.pallas import tpu as pltpu
from jax.experimental.pallas import tpu_sc as plsc

N_NODES = 100000
N_EDGES = 6400000
N_STEPS = 5
R = 0.025
THRESHOLD = 0.001378
B = 0.001
NOISE_SPARSITY = 1.5
NOISE_STD = 0.3
TAU = 0.01
DT = 0.0001

NC = 2
NT = 16
SL = 6400
N_PAD = NT * SL
SL2 = N_PAD // (NC * NT)
CHUNK = 2048
K = CHUNK // 128
CPT = 98
E_PAD = NC * NT * CPT * CHUNK
ROWS_PER_TILE = CPT * K


def _edge_body(act_hbm, src_hbm, dst_hbm, w_hbm, part_hbm,
               act_sp, acc_sp, src_v, dst_v, w_v, val_v, zbuf,
               lsem, gsem, ssem):
    core = lax.axis_index("c")
    sid = lax.axis_index("s")
    nslice = pl.ds(sid * SL, SL)

    def zinit(i, carry):
        zbuf[pl.ds(i * 16, 16)] = jnp.zeros((16,), jnp.float32)
        return carry
    lax.fori_loop(0, SL // 16, zinit, 0)

    pltpu.sync_copy(act_hbm.at[nslice], act_sp.at[nslice])
    pltpu.sync_copy(zbuf, acc_sp.at[nslice])
    plsc.subcore_barrier()

    row_base = (core * NT + sid) * ROWS_PER_TILE

    def fire_lin(c, p):
        r0 = row_base + c * K
        pltpu.async_copy(src_hbm.at[pl.ds(r0, K)], src_v.at[p], lsem)
        pltpu.async_copy(dst_hbm.at[pl.ds(r0, K)], dst_v.at[p], lsem)
        pltpu.async_copy(w_hbm.at[pl.ds(r0, K)], w_v.at[p], lsem)

    def wait_lin(c, p):
        r0 = row_base + c * K
        pltpu.make_async_copy(src_hbm.at[pl.ds(r0, K)],
                              src_v.at[p], lsem).wait()
        pltpu.make_async_copy(dst_hbm.at[pl.ds(r0, K)],
                              dst_v.at[p], lsem).wait()
        pltpu.make_async_copy(w_hbm.at[pl.ds(r0, K)],
                              w_v.at[p], lsem).wait()

    def wait_scat(p):
        pltpu.make_async_copy(w_hbm.at[pl.ds(0, K)], val_v.at[p], ssem).wait()

    fire_lin(0, 0)

    def group_body(g, carry):
        for b in range(2):
            p, q = b, 1 - b
            c = 2 * g + b
            wait_lin(c, p)
            for j in range(K):
                pltpu.async_copy(act_sp.at[src_v.at[p, j]],
                                 val_v.at[p, j], gsem)
            if b == 0:
                @pl.when(g > 0)
                def _():
                    wait_scat(q)
                fire_lin(c + 1, q)
            else:
                wait_scat(q)

                @pl.when(g < CPT // 2 - 1)
                def _():
                    fire_lin(c + 1, q)
            pltpu.make_async_copy(w_hbm.at[pl.ds(0, K)],
                                  val_v.at[p], gsem).wait()

            for j in range(K):
                for m in range(8):
                    s = pl.ds(m * 16, 16)
                    val_v[p, j, s] = val_v[p, j, s] * w_v[p, j, s]
            for j in range(K):
                pltpu.async_copy(val_v.at[p, j],
                                 acc_sp.at[dst_v.at[p, j]],
                                 ssem, add=True)
        return carry
    lax.fori_loop(0, CPT // 2, group_body, 0)
    wait_scat(1)
    plsc.subcore_barrier()
    pltpu.sync_copy(acc_sp.at[nslice],
                    part_hbm.at[pl.ds(core * N_PAD + sid * SL, SL)])


def _elem_body(part_hbm, bt_hbm, out_hbm, act_hbm,
               y0buf, y1buf, bbuf, sbuf, abuf):
    core = lax.axis_index("c")
    sid = lax.axis_index("s")
    wid = sid * NC + core
    base = wid * SL2

    pltpu.sync_copy(part_hbm.at[pl.ds(base, SL2)], y0buf)
    pltpu.sync_copy(part_hbm.at[pl.ds(N_PAD + base, SL2)], y1buf)
    pltpu.sync_copy(bt_hbm.at[pl.ds(base, SL2)], bbuf)

    def ew(i, carry):
        s = pl.ds(i * 16, 16)
        y = y0buf[s] + y1buf[s]
        l = R * y + bbuf[s]
        spk = jnp.where(l > THRESHOLD, jnp.float32(1.0), jnp.float32(0.0))
        sbuf[s] = spk
        abuf[s] = y + spk - y / TAU * DT
        return carry
    lax.fori_loop(0, SL2 // 16, ew, 0)

    pltpu.sync_copy(sbuf, out_hbm.at[pl.ds(base, SL2)])
    pltpu.sync_copy(abuf, act_hbm.at[pl.ds(base, SL2)])


_mesh = plsc.VectorSubcoreMesh(core_axis_name="c", subcore_axis_name="s")

_edge_call = pl.kernel(
    _edge_body,
    out_type=jax.ShapeDtypeStruct((NC * N_PAD,), jnp.float32),
    mesh=_mesh,
    scratch_types=[
        pltpu.VMEM_SHARED((N_PAD,), jnp.float32),
        pltpu.VMEM_SHARED((N_PAD,), jnp.float32),
        pltpu.VMEM((2, K, 128), jnp.int32),
        pltpu.VMEM((2, K, 128), jnp.int32),
        pltpu.VMEM((2, K, 128), jnp.float32),
        pltpu.VMEM((2, K, 128), jnp.float32),
        pltpu.VMEM((SL,), jnp.float32),
        pltpu.SemaphoreType.DMA,
        pltpu.SemaphoreType.DMA,
        pltpu.SemaphoreType.DMA,
    ],
)

_elem_call = pl.kernel(
    _elem_body,
    out_type=(jax.ShapeDtypeStruct((N_PAD,), jnp.float32),
              jax.ShapeDtypeStruct((N_PAD,), jnp.float32)),
    mesh=_mesh,
    scratch_types=[
        pltpu.VMEM((SL2,), jnp.float32),
        pltpu.VMEM((SL2,), jnp.float32),
        pltpu.VMEM((SL2,), jnp.float32),
        pltpu.VMEM((SL2,), jnp.float32),
        pltpu.VMEM((SL2,), jnp.float32),
    ],
)


def kernel(activation, weights, edge_index):
    act = jnp.pad(activation, (0, N_PAD - N_NODES))
    src = edge_index[0]
    dst = edge_index[1]
    pad = E_PAD - N_EDGES
    fill = (jnp.arange(pad, dtype=jnp.int32) * 997) % N_NODES
    src2 = jnp.concatenate([src, fill]).reshape(E_PAD // 128, 128)
    dst2 = jnp.concatenate([dst, fill]).reshape(E_PAD // 128, 128)
    w2 = jnp.concatenate(
        [weights, jnp.zeros((pad,), jnp.float32)]).reshape(E_PAD // 128, 128)

    noise_key = jax.random.key(42)
    spikes = []
    for t in range(N_STEPS):
        kt = jax.random.fold_in(noise_key, t)
        ka, kb = jax.random.split(kt)
        noise = NOISE_STD * jax.random.normal(ka, (N_NODES,), jnp.float32)
        filt = (jax.random.normal(kb, (N_NODES,), jnp.float32)
                > NOISE_SPARSITY).astype(jnp.float32)
        bt = jnp.pad(B * (1.0 + noise * filt), (0, N_PAD - N_NODES))
        part = _edge_call(act, src2, dst2, w2)
        spk, act = _elem_call(part, bt)
        spikes.append(spk[:N_NODES])
    return jnp.stack(spikes, axis=1)


# speedup vs baseline: 378.1625x; 1.2178x over previous
import jax
import jax.numpy as jnp
from jax import lax
from jax.experimental import pallas as pl
from jax.experimental.pallas import tpu as pltpu
from jax.experimental.pallas import tpu_sc as plsc

N_NODES = 100000
N_EDGES = 6400000
N_STEPS = 5
R = 0.025
THRESHOLD = 0.001378
B = 0.001
NOISE_SPARSITY = 1.5
NOISE_STD = 0.3
TAU = 0.01
DT = 0.0001

NC = 2
NT = 16
SL = 6400
N_PAD = NT * SL
SL2 = N_PAD // (NC * NT)
CHUNK = 2048
K = CHUNK // 128
CPT = 98
E_PAD = NC * NT * CPT * CHUNK
ROWS_PER_TILE = CPT * K


def _edge_body(act_hbm, src_hbm, dst_hbm, w_hbm, zero_hbm, part_hbm,
               acc_sp, act_tile, src_v, dst_v, w_v, val_v,
               lsem, ssem):
    core = lax.axis_index("c")
    sid = lax.axis_index("s")

    @pl.when(sid == 0)
    def _():
        pltpu.sync_copy(zero_hbm, acc_sp)
    plsc.subcore_barrier()

    row_base = (core * NT + sid) * ROWS_PER_TILE

    def fire_lin(c, p):
        ridx = row_base + c * K + lax.iota(jnp.int32, 16)
        pltpu.async_copy(src_hbm.at[ridx], src_v.at[p], lsem)
        pltpu.async_copy(dst_hbm.at[ridx], dst_v.at[p], lsem)
        pltpu.async_copy(w_hbm.at[ridx], w_v.at[p], lsem)

    def wait_lin(p):
        pltpu.make_async_copy(src_hbm.at[lax.iota(jnp.int32, 16)],
                              src_v.at[p], lsem).wait()
        pltpu.make_async_copy(dst_hbm.at[lax.iota(jnp.int32, 16)],
                              dst_v.at[p], lsem).wait()
        pltpu.make_async_copy(w_hbm.at[lax.iota(jnp.int32, 16)],
                              w_v.at[p], lsem).wait()

    def wait_scat(p):
        pltpu.make_async_copy(w_hbm.at[lax.iota(jnp.int32, 16)],
                              val_v.at[p], ssem).wait()

    fire_lin(0, 0)
    pltpu.sync_copy(act_hbm, act_tile)

    def group_body(g, carry):
        for b in range(2):
            p, q = b, 1 - b
            c = 2 * g + b
            wait_lin(p)
            if b == 0:
                @pl.when(g > 0)
                def _():
                    wait_scat(q)
                fire_lin(c + 1, q)
            else:
                wait_scat(q)

                @pl.when(g < CPT // 2 - 1)
                def _():
                    fire_lin(c + 1, q)

            for j in range(K):
                for m in range(8):
                    s = pl.ds(m * 16, 16)
                    v = plsc.load_gather(act_tile, [src_v[p, j, s]])
                    val_v[p, j, s] = v * w_v[p, j, s]
            for j in range(K):
                pltpu.async_copy(val_v.at[p, j],
                                 acc_sp.at[dst_v.at[p, j]],
                                 ssem, add=True)
        return carry
    lax.fori_loop(0, CPT // 2, group_body, 0)
    wait_scat(1)
    plsc.subcore_barrier()

    @pl.when(jnp.logical_and(core == 0, sid == 0))
    def _():
        pltpu.sync_copy(acc_sp, part_hbm.at[pl.ds(0, N_PAD)])

    @pl.when(jnp.logical_and(core == 1, sid == 0))
    def _():
        pltpu.sync_copy(acc_sp, part_hbm.at[pl.ds(N_PAD, N_PAD)])


def _elem_body(part_hbm, bt_hbm, out_hbm, act_hbm,
               y0buf, y1buf, bbuf, sbuf, abuf):
    core = lax.axis_index("c")
    sid = lax.axis_index("s")
    wid = sid * NC + core
    base = wid * SL2

    pltpu.sync_copy(part_hbm.at[pl.ds(base, SL2)], y0buf)
    pltpu.sync_copy(part_hbm.at[pl.ds(N_PAD + base, SL2)], y1buf)
    pltpu.sync_copy(bt_hbm.at[pl.ds(base, SL2)], bbuf)

    def ew(i, carry):
        s = pl.ds(i * 16, 16)
        y = y0buf[s] + y1buf[s]
        l = R * y + bbuf[s]
        spk = jnp.where(l > THRESHOLD, jnp.float32(1.0), jnp.float32(0.0))
        sbuf[s] = spk
        abuf[s] = y + spk - y / TAU * DT
        return carry
    lax.fori_loop(0, SL2 // 16, ew, 0)

    pltpu.sync_copy(sbuf, out_hbm.at[pl.ds(base, SL2)])
    pltpu.sync_copy(abuf, act_hbm.at[pl.ds(base, SL2)])


_mesh = plsc.VectorSubcoreMesh(core_axis_name="c", subcore_axis_name="s")

_edge_call = pl.kernel(
    _edge_body,
    out_type=jax.ShapeDtypeStruct((NC * N_PAD,), jnp.float32),
    mesh=_mesh,
    compiler_params=pltpu.CompilerParams(needs_layout_passes=False),
    scratch_types=[
        pltpu.VMEM_SHARED((N_PAD,), jnp.float32),
        pltpu.VMEM((N_PAD,), jnp.float32),
        pltpu.VMEM((2, K, 128), jnp.int32),
        pltpu.VMEM((2, K, 128), jnp.int32),
        pltpu.VMEM((2, K, 128), jnp.float32),
        pltpu.VMEM((2, K, 128), jnp.float32),
        pltpu.SemaphoreType.DMA,
        pltpu.SemaphoreType.DMA,
    ],
)

_elem_call = pl.kernel(
    _elem_body,
    out_type=(jax.ShapeDtypeStruct((N_PAD,), jnp.float32),
              jax.ShapeDtypeStruct((N_PAD,), jnp.float32)),
    mesh=_mesh,
    scratch_types=[
        pltpu.VMEM((SL2,), jnp.float32),
        pltpu.VMEM((SL2,), jnp.float32),
        pltpu.VMEM((SL2,), jnp.float32),
        pltpu.VMEM((SL2,), jnp.float32),
        pltpu.VMEM((SL2,), jnp.float32),
    ],
)


def kernel(activation, weights, edge_index):
    act = jnp.pad(activation, (0, N_PAD - N_NODES))
    src = edge_index[0]
    dst = edge_index[1]
    pad = E_PAD - N_EDGES
    fill = (jnp.arange(pad, dtype=jnp.int32) * 997) % N_NODES
    src2 = jnp.concatenate([src, fill]).reshape(E_PAD // 128, 128)
    dst2 = jnp.concatenate([dst, fill]).reshape(E_PAD // 128, 128)
    w2 = jnp.concatenate(
        [weights, jnp.zeros((pad,), jnp.float32)]).reshape(E_PAD // 128, 128)

    zero = jnp.zeros((N_PAD,), jnp.float32)
    noise_key = jax.random.key(42)
    spikes = []
    for t in range(N_STEPS):
        kt = jax.random.fold_in(noise_key, t)
        ka, kb = jax.random.split(kt)
        noise = NOISE_STD * jax.random.normal(ka, (N_NODES,), jnp.float32)
        filt = (jax.random.normal(kb, (N_NODES,), jnp.float32)
                > NOISE_SPARSITY).astype(jnp.float32)
        bt = jnp.pad(B * (1.0 + noise * filt), (0, N_PAD - N_NODES))
        part = _edge_call(act, src2, dst2, w2, zero)
        spk, act = _elem_call(part, bt)
        spikes.append(spk[:N_NODES])
    return jnp.stack(spikes, axis=1)
